# baseline (speedup 1.0000x reference)
import numpy as np
import jax
import jax.numpy as jnp
from jax.experimental import pallas as pl

_HI = jax.lax.Precision.HIGHEST
_E = 72
_N = 24
_ITERS = 10


def _graph_constants():
    pcm = np.zeros((12, 24), dtype=np.int64)
    for r in range(4):
        pcm[r, 6 * r:6 * r + 6] = 1
    for j in range(24):
        pcm[4 + (j % 4), j] = 1
        pcm[8 + ((j + j // 6) % 4), j] = 1
    rolled = np.stack(np.where(pcm), axis=1)
    var = rolled[:, 1]
    G24 = (np.arange(24)[:, None] == var[None, :]).astype(np.float32)
    V = (var[:, None] == var[None, :]).astype(np.float32)
    return G24, V


_G24, _VM = _graph_constants()


def _elu(x):
    return jnp.where(x > 0, x, jnp.exp(jnp.minimum(x, 0.0)) - 1.0)


def _norm_by_mean(x):
    mean = jnp.mean(x, axis=1, keepdims=True)
    safe = jnp.where(mean == 0, 1.0, mean)
    return jnp.where(mean == 0, 0.0, x / safe)


def _group_roll(x, d, lane):
    a = jnp.concatenate([x[:, d:], x[:, :d]], axis=1)
    b = jnp.concatenate([x[:, _E - (6 - d):], x[:, :_E - (6 - d)]], axis=1)
    return jnp.where(lane < 6 - d, a, b)


def _estep(h_m, h_e_old, h_m_res, llr_res):
    lane = jax.lax.broadcasted_iota(jnp.int32, (1, _E), 1) % 6
    t = jnp.tanh(h_m * 0.5)
    prod = _group_roll(t, 1, lane)
    for d in range(2, 6):
        prod = prod * _group_roll(t, d, lane)
    p = jnp.clip(prod, -0.999999, 0.999999)
    h_e_new = jnp.log((1.0 + p) / (1.0 - p))
    h_e_res = jnp.abs(h_e_new - h_e_old)
    f0 = _norm_by_mean(jnp.abs(h_e_new))
    f1 = _norm_by_mean(h_e_res)
    f2 = _norm_by_mean(h_m_res)
    f3 = _norm_by_mean(llr_res)
    return h_e_new, f0, f1, f2, f3


def _e0_body(llr_ref, g24_ref, hr_ref, hen_ref, f0_ref, f1_ref, f2_ref, f3_ref):
    h_r = jnp.dot(llr_ref[...], g24_ref[...],
                  preferred_element_type=jnp.float32, precision=_HI)
    z = jnp.zeros_like(h_r)
    h_e_new, f0, f1, f2, f3 = _estep(h_r, z, z, z)
    hr_ref[...] = h_r
    hen_ref[...] = h_e_new
    f0_ref[...] = f0
    f1_ref[...] = f1
    f2_ref[...] = f2
    f3_ref[...] = f3


def _mlp_body(x_ref, w1_ref, b1_ref, w2_ref, b2_ref, w3_ref, b3_ref, out_ref):
    X = x_ref[...]
    feat = X[:, :4]
    hen = X[:, 4:5]
    x1 = _elu(jnp.dot(feat, w1_ref[...],
                      preferred_element_type=jnp.float32, precision=_HI) + b1_ref[...])
    x2 = _elu(jnp.dot(x1, w2_ref[...],
                      preferred_element_type=jnp.float32, precision=_HI) + b2_ref[...])
    w = _elu(jnp.dot(x2, w3_ref[...],
                     preferred_element_type=jnp.float32, precision=_HI) + b3_ref[...])
    out_ref[...] = w * hen


def _ve_body(hew_ref, hr_ref, hm_ref, llrg_ref, v_ref,
             hm2_ref, llrg2_ref, hen_ref, f0_ref, f1_ref, f2_ref, f3_ref):
    h_e_w = hew_ref[...]
    h_r = hr_ref[...]
    vt = jnp.dot(h_e_w, v_ref[...],
                 preferred_element_type=jnp.float32, precision=_HI)
    llr_g_new = vt + h_r
    h_m_new = llr_g_new - h_e_w
    h_m_res = jnp.abs(h_m_new - hm_ref[...])
    llr_res = jnp.abs(llr_g_new - llrg_ref[...])
    h_e_new, f0, f1, f2, f3 = _estep(h_m_new, h_e_w, h_m_res, llr_res)
    hm2_ref[...] = h_m_new
    llrg2_ref[...] = llr_g_new
    hen_ref[...] = h_e_new
    f0_ref[...] = f0
    f1_ref[...] = f1
    f2_ref[...] = f2
    f3_ref[...] = f3


def _vfinal_body(hew_ref, hr_ref, v_ref, out_ref):
    vt = jnp.dot(hew_ref[...], v_ref[...],
                 preferred_element_type=jnp.float32, precision=_HI)
    out_ref[...] = (vt + hr_ref[...])[:, :_N]


def kernel(llr, W1, b1, W2, b2, W3, b3):
    B = llr.shape[0]
    Bt = min(1024, B)
    Nrows = B * _E
    NT = min(8192, Nrows)
    g24 = jnp.asarray(_G24)
    vM = jnp.asarray(_VM)
    b1r = b1.reshape(1, 32)
    b2r = b2.reshape(1, 32)
    b3r = b3.reshape(1, 1)

    be = lambda: pl.BlockSpec((Bt, _E), lambda i: (i, 0))
    const = lambda shape: pl.BlockSpec(shape, lambda i: (0, 0))
    f32 = jnp.float32
    be_shape = jax.ShapeDtypeStruct((B, _E), f32)

    e0 = pl.pallas_call(
        _e0_body,
        grid=(B // Bt,),
        in_specs=[pl.BlockSpec((Bt, _N), lambda i: (i, 0)), const((_N, _E))],
        out_specs=tuple(be() for _ in range(6)),
        out_shape=tuple(be_shape for _ in range(6)),
    )

    mlp = pl.pallas_call(
        _mlp_body,
        grid=(Nrows // NT,),
        in_specs=[
            pl.BlockSpec((NT, 5), lambda i: (i, 0)),
            const((4, 32)), const((1, 32)),
            const((32, 32)), const((1, 32)),
            const((32, 1)), const((1, 1)),
        ],
        out_specs=pl.BlockSpec((NT, 1), lambda i: (i, 0)),
        out_shape=jax.ShapeDtypeStruct((Nrows, 1), f32),
    )

    ve = pl.pallas_call(
        _ve_body,
        grid=(B // Bt,),
        in_specs=[be(), be(), be(), be(), const((_E, _E))],
        out_specs=tuple(be() for _ in range(7)),
        out_shape=tuple(be_shape for _ in range(7)),
    )

    vfinal = pl.pallas_call(
        _vfinal_body,
        grid=(B // Bt,),
        in_specs=[be(), be(), const((_E, _E))],
        out_specs=pl.BlockSpec((Bt, _N), lambda i: (i, 0)),
        out_shape=jax.ShapeDtypeStruct((B, _N), f32),
    )

    h_r, hen, f0, f1, f2, f3 = e0(llr, g24)
    hm = h_r
    llrg = h_r
    out = None
    for i in range(_ITERS):
        X = jnp.stack([f0, f1, f2, f3, hen], axis=-1).reshape(Nrows, 5)
        hew = mlp(X, W1, b1r, W2, b2r, W3, b3r).reshape(B, _E)
        if i < _ITERS - 1:
            hm, llrg, hen, f0, f1, f2, f3 = ve(hew, h_r, hm, llrg, vM)
        else:
            out = vfinal(hew, h_r, vM)
    return out


# speedup vs baseline: 5.0305x; 5.0305x over previous
import numpy as np
import jax
import jax.numpy as jnp
from jax.experimental import pallas as pl

_HI = jax.lax.Precision.HIGHEST
_E = 72
_N = 24
_ITERS = 10


def _graph_constants():
    pcm = np.zeros((12, 24), dtype=np.int64)
    for r in range(4):
        pcm[r, 6 * r:6 * r + 6] = 1
    for j in range(24):
        pcm[4 + (j % 4), j] = 1
        pcm[8 + ((j + j // 6) % 4), j] = 1
    rolled = np.stack(np.where(pcm), axis=1)
    var = rolled[:, 1]
    G24 = (np.arange(24)[:, None] == var[None, :]).astype(np.float32)
    V = (var[:, None] == var[None, :]).astype(np.float32)
    return G24, V


_G24, _VM = _graph_constants()


def _elu(x):
    return jnp.where(x > 0, x, jnp.exp(jnp.minimum(x, 0.0)) - 1.0)


def _norm_by_mean(x):
    mean = jnp.mean(x, axis=1, keepdims=True)
    safe = jnp.where(mean == 0, 1.0, mean)
    return jnp.where(mean == 0, 0.0, x / safe)


def _group_roll(x, d, lane):
    a = jnp.concatenate([x[:, d:], x[:, :d]], axis=1)
    b = jnp.concatenate([x[:, _E - (6 - d):], x[:, :_E - (6 - d)]], axis=1)
    return jnp.where(lane < 6 - d, a, b)


def _estep(h_m, h_e_old, h_m_res, llr_res):
    lane = jax.lax.broadcasted_iota(jnp.int32, (1, _E), 1) % 6
    t = jnp.tanh(h_m * 0.5)
    prod = _group_roll(t, 1, lane)
    for d in range(2, 6):
        prod = prod * _group_roll(t, d, lane)
    p = jnp.clip(prod, -0.999999, 0.999999)
    h_e_new = jnp.log((1.0 + p) / (1.0 - p))
    h_e_res = jnp.abs(h_e_new - h_e_old)
    f0 = _norm_by_mean(jnp.abs(h_e_new))
    f1 = _norm_by_mean(h_e_res)
    f2 = _norm_by_mean(h_m_res)
    f3 = _norm_by_mean(llr_res)
    return h_e_new, f0, f1, f2, f3


def _e0_body(llr_ref, g24_ref, hr_ref, hen_ref, f0_ref, f1_ref, f2_ref, f3_ref):
    h_r = jnp.dot(llr_ref[...], g24_ref[...],
                  preferred_element_type=jnp.float32, precision=_HI)
    z = jnp.zeros_like(h_r)
    h_e_new, f0, f1, f2, f3 = _estep(h_r, z, z, z)
    hr_ref[...] = h_r
    hen_ref[...] = h_e_new
    f0_ref[...] = f0
    f1_ref[...] = f1
    f2_ref[...] = f2
    f3_ref[...] = f3


def _mlp_body(x_ref, w1_ref, b1_ref, w2_ref, b2_ref, w3_ref, b3_ref, out_ref):
    X = x_ref[...]
    feat = X[:4, :]
    hen = X[4:5, :]
    x1 = _elu(jnp.dot(w1_ref[...], feat,
                      preferred_element_type=jnp.float32, precision=_HI) + b1_ref[...])
    x2 = _elu(jnp.dot(w2_ref[...], x1,
                      preferred_element_type=jnp.float32, precision=_HI) + b2_ref[...])
    w = _elu(jnp.dot(w3_ref[...], x2,
                     preferred_element_type=jnp.float32, precision=_HI) + b3_ref[...])
    out_ref[...] = w * hen


def _ve_body(hew_ref, hr_ref, hm_ref, llrg_ref, v_ref,
             hm2_ref, llrg2_ref, hen_ref, f0_ref, f1_ref, f2_ref, f3_ref):
    h_e_w = hew_ref[...]
    h_r = hr_ref[...]
    vt = jnp.dot(h_e_w, v_ref[...],
                 preferred_element_type=jnp.float32, precision=_HI)
    llr_g_new = vt + h_r
    h_m_new = llr_g_new - h_e_w
    h_m_res = jnp.abs(h_m_new - hm_ref[...])
    llr_res = jnp.abs(llr_g_new - llrg_ref[...])
    h_e_new, f0, f1, f2, f3 = _estep(h_m_new, h_e_w, h_m_res, llr_res)
    hm2_ref[...] = h_m_new
    llrg2_ref[...] = llr_g_new
    hen_ref[...] = h_e_new
    f0_ref[...] = f0
    f1_ref[...] = f1
    f2_ref[...] = f2
    f3_ref[...] = f3


def _vfinal_body(hew_ref, hr_ref, v_ref, out_ref):
    vt = jnp.dot(hew_ref[...], v_ref[...],
                 preferred_element_type=jnp.float32, precision=_HI)
    out_ref[...] = (vt + hr_ref[...])[:, :_N]


def kernel(llr, W1, b1, W2, b2, W3, b3):
    B = llr.shape[0]
    Bt = min(1024, B)
    Nrows = B * _E
    NT = min(32768, Nrows)
    g24 = jnp.asarray(_G24)
    vM = jnp.asarray(_VM)
    w1t = W1.T
    w2t = W2.T
    w3t = W3.T
    b1c = b1.reshape(32, 1)
    b2c = b2.reshape(32, 1)
    b3c = b3.reshape(1, 1)

    be = lambda: pl.BlockSpec((Bt, _E), lambda i: (i, 0))
    const = lambda shape: pl.BlockSpec(shape, lambda i: (0, 0))
    f32 = jnp.float32
    be_shape = jax.ShapeDtypeStruct((B, _E), f32)

    e0 = pl.pallas_call(
        _e0_body,
        grid=(B // Bt,),
        in_specs=[pl.BlockSpec((Bt, _N), lambda i: (i, 0)), const((_N, _E))],
        out_specs=tuple(be() for _ in range(6)),
        out_shape=tuple(be_shape for _ in range(6)),
    )

    mlp = pl.pallas_call(
        _mlp_body,
        grid=(Nrows // NT,),
        in_specs=[
            pl.BlockSpec((5, NT), lambda i: (0, i)),
            const((32, 4)), const((32, 1)),
            const((32, 32)), const((32, 1)),
            const((1, 32)), const((1, 1)),
        ],
        out_specs=pl.BlockSpec((1, NT), lambda i: (0, i)),
        out_shape=jax.ShapeDtypeStruct((1, Nrows), f32),
    )

    ve = pl.pallas_call(
        _ve_body,
        grid=(B // Bt,),
        in_specs=[be(), be(), be(), be(), const((_E, _E))],
        out_specs=tuple(be() for _ in range(7)),
        out_shape=tuple(be_shape for _ in range(7)),
    )

    vfinal = pl.pallas_call(
        _vfinal_body,
        grid=(B // Bt,),
        in_specs=[be(), be(), const((_E, _E))],
        out_specs=pl.BlockSpec((Bt, _N), lambda i: (i, 0)),
        out_shape=jax.ShapeDtypeStruct((B, _N), f32),
    )

    h_r, hen, f0, f1, f2, f3 = e0(llr, g24)
    hm = h_r
    llrg = h_r
    out = None
    for i in range(_ITERS):
        X = jnp.stack([f0.reshape(Nrows), f1.reshape(Nrows), f2.reshape(Nrows),
                       f3.reshape(Nrows), hen.reshape(Nrows)], axis=0)
        hew = mlp(X, w1t, b1c, w2t, b2c, w3t, b3c).reshape(B, _E)
        if i < _ITERS - 1:
            hm, llrg, hen, f0, f1, f2, f3 = ve(hew, h_r, hm, llrg, vM)
        else:
            out = vfinal(hew, h_r, vM)
    return out
